# Initial kernel scaffold; baseline (speedup 1.0000x reference)
import jax
import jax.numpy as jnp
from jax.experimental import pallas as pl


def kernel(pred, gt):
    raise NotImplementedError("write your pallas kernel here")


# speedup vs baseline: 1.3530x; 1.3530x over previous
import functools

import jax
import jax.numpy as jnp
from jax.experimental import pallas as pl

_K = 10
_R = 256


def _cov_body(ptsT8_ref, qn_ref, p_ref, out_ref):
    q8 = ptsT8_ref[0]
    p = p_ref[0]
    qn = qn_ref[0]

    sq = jnp.sum(q8 * q8, axis=0, keepdims=True)
    pq = jax.lax.dot_general(
        p, q8, (((1,), (0,)), ((), ())),
        preferred_element_type=jnp.float32,
        precision=jax.lax.Precision.HIGHEST)
    scores = sq - 2.0 * pq

    msum = jnp.zeros_like(scores)
    for _ in range(_K):
        m = jnp.min(scores, axis=1, keepdims=True)
        msk = scores == m
        msum = jnp.where(msk, 1.0, msum)
        scores = jnp.where(msk, jnp.inf, scores)

    x = qn[:, 0:1]
    y = qn[:, 1:2]
    z = qn[:, 2:3]
    zq = jnp.zeros_like(x)
    q2 = jnp.concatenate([x * x, y * y, z * z, x * y, x * z, y * z, zq, zq],
                         axis=1)
    s1 = jax.lax.dot_general(
        msum, qn, (((1,), (0,)), ((), ())),
        preferred_element_type=jnp.float32,
        precision=jax.lax.Precision.HIGHEST)
    s2 = jax.lax.dot_general(
        msum, q2, (((1,), (0,)), ((), ())),
        preferred_element_type=jnp.float32,
        precision=jax.lax.Precision.HIGHEST)

    inv_k = 1.0 / _K
    mux = s1[:, 0:1] * inv_k
    muy = s1[:, 1:2] * inv_k
    muz = s1[:, 2:3] * inv_k
    cxx = s2[:, 0:1] * inv_k - mux * mux
    cyy = s2[:, 1:2] * inv_k - muy * muy
    czz = s2[:, 2:3] * inv_k - muz * muz
    cxy = s2[:, 3:4] * inv_k - mux * muy
    cxz = s2[:, 4:5] * inv_k - mux * muz
    cyz = s2[:, 5:6] * inv_k - muy * muz
    z0 = jnp.zeros_like(cxx)
    out_ref[0] = jnp.concatenate([cxx, cyy, czz, cxy, cxz, cyz, z0, z0],
                                 axis=1)


@functools.partial(jax.jit, static_argnames=("interpret",))
def _normals_cov(pts, interpret=False):
    g, _, n = pts.shape
    pts_t8 = jnp.concatenate(
        [pts, jnp.zeros((g, 5, n), dtype=pts.dtype)], axis=1)
    pts_n8 = jnp.transpose(pts_t8, (0, 2, 1))
    grid = (g, n // _R)
    return pl.pallas_call(
        _cov_body,
        grid=grid,
        in_specs=[
            pl.BlockSpec((1, 8, n), lambda b, i: (b, 0, 0)),
            pl.BlockSpec((1, n, 8), lambda b, i: (b, 0, 0)),
            pl.BlockSpec((1, _R, 8), lambda b, i: (b, i, 0)),
        ],
        out_specs=pl.BlockSpec((1, _R, 8), lambda b, i: (b, i, 0)),
        out_shape=jax.ShapeDtypeStruct((g, n, 8), jnp.float32),
        interpret=interpret,
    )(pts_t8, pts_n8, pts_n8)


def _kernel_impl(pred, gt, interpret=False):
    b = pred.shape[0]
    pts = jnp.concatenate([pred, gt], axis=0)
    c = _normals_cov(pts, interpret=interpret)
    cxx, cyy, czz = c[..., 0], c[..., 1], c[..., 2]
    cxy, cxz, cyz = c[..., 3], c[..., 4], c[..., 5]
    row0 = jnp.stack([cxx, cxy, cxz], axis=-1)
    row1 = jnp.stack([cxy, cyy, cyz], axis=-1)
    row2 = jnp.stack([cxz, cyz, czz], axis=-1)
    cov = jnp.stack([row0, row1, row2], axis=-2)
    cov = cov + 1e-8 * jnp.eye(3, dtype=cov.dtype)
    _, evecs = jnp.linalg.eigh(cov)
    normals = evecs[..., :, 0]
    return jnp.mean((normals[:b] - normals[b:]) ** 2)


def kernel(pred, gt):
    return _kernel_impl(pred, gt)


# speedup vs baseline: 71.6505x; 52.9578x over previous
import functools

import jax
import jax.numpy as jnp
from jax.experimental import pallas as pl

_K = 10
_R = 256
_SWEEPS = 6


def _cov_body(ptsT8_ref, qn_ref, p_ref, out_ref):
    q8 = ptsT8_ref[0]
    p = p_ref[0]
    qn = qn_ref[0]

    sq = jnp.sum(q8 * q8, axis=0, keepdims=True)
    pq = jax.lax.dot_general(
        p, q8, (((1,), (0,)), ((), ())),
        preferred_element_type=jnp.float32)
    scores = sq - 2.0 * pq

    msum = jnp.zeros_like(scores)
    for _ in range(_K):
        m = jnp.min(scores, axis=1, keepdims=True)
        msk = scores == m
        msum = jnp.where(msk, 1.0, msum)
        scores = jnp.where(msk, jnp.inf, scores)

    x = qn[:, 0:1]
    y = qn[:, 1:2]
    z = qn[:, 2:3]
    zq = jnp.zeros_like(x)
    q2 = jnp.concatenate([x * x, y * y, z * z, x * y, x * z, y * z, zq, zq],
                         axis=1)
    s1 = jax.lax.dot_general(
        msum, qn, (((1,), (0,)), ((), ())),
        preferred_element_type=jnp.float32,
        precision=jax.lax.Precision.HIGHEST)
    s2 = jax.lax.dot_general(
        msum, q2, (((1,), (0,)), ((), ())),
        preferred_element_type=jnp.float32,
        precision=jax.lax.Precision.HIGHEST)

    inv_k = 1.0 / _K
    mux = s1[:, 0:1] * inv_k
    muy = s1[:, 1:2] * inv_k
    muz = s1[:, 2:3] * inv_k
    cxx = s2[:, 0:1] * inv_k - mux * mux
    cyy = s2[:, 1:2] * inv_k - muy * muy
    czz = s2[:, 2:3] * inv_k - muz * muz
    cxy = s2[:, 3:4] * inv_k - mux * muy
    cxz = s2[:, 4:5] * inv_k - mux * muz
    cyz = s2[:, 5:6] * inv_k - muy * muz
    z0 = jnp.zeros_like(cxx)
    out_ref[0] = jnp.concatenate([cxx, cyy, czz, cxy, cxz, cyz, z0, z0],
                                 axis=1)


def _hk(i, j):
    return (min(i, j), max(i, j))


def _jacobi_rotate(h, v, p, q):
    r = 3 - p - q
    a = h[_hk(p, p)]
    b = h[_hk(q, q)]
    d = h[_hk(p, q)]
    e = h[_hk(p, r)]
    f = h[_hk(q, r)]
    tau = (b - a) / (2.0 * d)
    t = jnp.sign(tau) / (jnp.abs(tau) + jnp.sqrt(1.0 + tau * tau))
    t = jnp.where(d == 0.0, 0.0, t)
    c = jax.lax.rsqrt(1.0 + t * t)
    s = t * c
    cc = c * c
    ss = s * s
    sc2 = 2.0 * s * c
    h[_hk(p, p)] = cc * a - sc2 * d + ss * b
    h[_hk(q, q)] = ss * a + sc2 * d + cc * b
    h[_hk(p, q)] = s * c * (a - b) + (cc - ss) * d
    h[_hk(p, r)] = c * e - s * f
    h[_hk(q, r)] = s * e + c * f
    for i in range(3):
        vp = v[(i, p)]
        vq = v[(i, q)]
        v[(i, p)] = c * vp - s * vq
        v[(i, q)] = s * vp + c * vq


def _eig_mse_body(cov_ref, out_ref):
    h = {}
    h[(0, 0)] = cov_ref[0]
    h[(1, 1)] = cov_ref[1]
    h[(2, 2)] = cov_ref[2]
    h[(0, 1)] = cov_ref[3]
    h[(0, 2)] = cov_ref[4]
    h[(1, 2)] = cov_ref[5]
    one = jnp.ones_like(h[(0, 0)])
    zero = jnp.zeros_like(one)
    v = {}
    for i in range(3):
        for j in range(3):
            v[(i, j)] = one if i == j else zero

    for _ in range(_SWEEPS):
        _jacobi_rotate(h, v, 0, 2)
        _jacobi_rotate(h, v, 2, 1)
        _jacobi_rotate(h, v, 0, 1)

    w0, w1, w2 = h[(0, 0)], h[(1, 1)], h[(2, 2)]
    sel0 = jnp.logical_and(w0 <= w1, w0 <= w2)
    sel1 = jnp.logical_and(w1 < w0, w1 <= w2)
    n = []
    for i in range(3):
        n.append(jnp.where(sel0, v[(i, 0)],
                           jnp.where(sel1, v[(i, 1)], v[(i, 2)])))

    acc = jnp.zeros_like(n[0][:8, :])
    for i in range(3):
        dlt = n[i][:8, :] - n[i][8:, :]
        acc = acc + dlt * dlt
    out_ref[...] = jnp.zeros((8, 128), jnp.float32) + jnp.sum(acc)


@jax.jit
def _normals_mse(pts):
    g, _, n = pts.shape
    pts_t8 = jnp.concatenate(
        [pts, jnp.zeros((g, 5, n), dtype=pts.dtype)], axis=1)
    pts_n8 = jnp.transpose(pts_t8, (0, 2, 1))
    covs = pl.pallas_call(
        _cov_body,
        grid=(g, n // _R),
        in_specs=[
            pl.BlockSpec((1, 8, n), lambda b, i: (b, 0, 0)),
            pl.BlockSpec((1, n, 8), lambda b, i: (b, 0, 0)),
            pl.BlockSpec((1, _R, 8), lambda b, i: (b, i, 0)),
        ],
        out_specs=pl.BlockSpec((1, _R, 8), lambda b, i: (b, i, 0)),
        out_shape=jax.ShapeDtypeStruct((g, n, 8), jnp.float32),
    )(pts_t8, pts_n8, pts_n8)

    total = g * n
    cov6 = jnp.transpose(covs, (2, 0, 1)).reshape(8, 16, total // 16)[:6]
    sq_sum = pl.pallas_call(
        _eig_mse_body,
        out_shape=jax.ShapeDtypeStruct((8, 128), jnp.float32),
    )(cov6)
    return sq_sum[0, 0]


def kernel(pred, gt):
    b, c, n = pred.shape
    pts = jnp.concatenate([pred, gt], axis=0)
    return _normals_mse(pts) / (b * c * n)


# speedup vs baseline: 99.9652x; 1.3952x over previous
import functools

import jax
import jax.numpy as jnp
from jax.experimental import pallas as pl

_K = 10
_R = 512
_SWEEPS = 6


def _cov_body(ptsT8_ref, qn_ref, p_ref, out_ref):
    q8 = ptsT8_ref[0]
    p = p_ref[0]
    qn = qn_ref[0]

    sq = jnp.sum(q8 * q8, axis=0, keepdims=True)
    pq = jax.lax.dot_general(
        p, q8, (((1,), (0,)), ((), ())),
        preferred_element_type=jnp.float32)
    scores = sq - 2.0 * pq

    for _ in range(_K):
        m = jnp.min(scores, axis=1, keepdims=True)
        scores = jnp.where(scores == m, jnp.inf, scores)
    msum = jnp.where(jnp.isinf(scores), 1.0, 0.0)

    x = qn[:, 0:1]
    y = qn[:, 1:2]
    z = qn[:, 2:3]
    zq = jnp.zeros_like(x)
    q12 = jnp.concatenate(
        [x, y, z, x * x, y * y, z * z, x * y, x * z, y * z,
         zq, zq, zq, zq, zq, zq, zq], axis=1)
    s12 = jax.lax.dot_general(
        msum, q12, (((1,), (0,)), ((), ())),
        preferred_element_type=jnp.float32,
        precision=jax.lax.Precision.HIGHEST)

    inv_k = 1.0 / _K
    mux = s12[:, 0:1] * inv_k
    muy = s12[:, 1:2] * inv_k
    muz = s12[:, 2:3] * inv_k
    cxx = s12[:, 3:4] * inv_k - mux * mux
    cyy = s12[:, 4:5] * inv_k - muy * muy
    czz = s12[:, 5:6] * inv_k - muz * muz
    cxy = s12[:, 6:7] * inv_k - mux * muy
    cxz = s12[:, 7:8] * inv_k - mux * muz
    cyz = s12[:, 8:9] * inv_k - muy * muz
    z0 = jnp.zeros_like(cxx)
    out_ref[0] = jnp.concatenate([cxx, cyy, czz, cxy, cxz, cyz, z0, z0],
                                 axis=1)


def _hk(i, j):
    return (min(i, j), max(i, j))


def _jacobi_rotate(h, v, p, q):
    r = 3 - p - q
    a = h[_hk(p, p)]
    b = h[_hk(q, q)]
    d = h[_hk(p, q)]
    e = h[_hk(p, r)]
    f = h[_hk(q, r)]
    tau = (b - a) / (2.0 * d)
    t = jnp.sign(tau) / (jnp.abs(tau) + jnp.sqrt(1.0 + tau * tau))
    t = jnp.where(d == 0.0, 0.0, t)
    c = jax.lax.rsqrt(1.0 + t * t)
    s = t * c
    cc = c * c
    ss = s * s
    sc2 = 2.0 * s * c
    h[_hk(p, p)] = cc * a - sc2 * d + ss * b
    h[_hk(q, q)] = ss * a + sc2 * d + cc * b
    h[_hk(p, q)] = s * c * (a - b) + (cc - ss) * d
    h[_hk(p, r)] = c * e - s * f
    h[_hk(q, r)] = s * e + c * f
    for i in range(3):
        vp = v[(i, p)]
        vq = v[(i, q)]
        v[(i, p)] = c * vp - s * vq
        v[(i, q)] = s * vp + c * vq


def _eig_mse_body(cov_ref, out_ref):
    h = {}
    h[(0, 0)] = cov_ref[0]
    h[(1, 1)] = cov_ref[1]
    h[(2, 2)] = cov_ref[2]
    h[(0, 1)] = cov_ref[3]
    h[(0, 2)] = cov_ref[4]
    h[(1, 2)] = cov_ref[5]
    one = jnp.ones_like(h[(0, 0)])
    zero = jnp.zeros_like(one)
    v = {}
    for i in range(3):
        for j in range(3):
            v[(i, j)] = one if i == j else zero

    for _ in range(_SWEEPS):
        _jacobi_rotate(h, v, 0, 2)
        _jacobi_rotate(h, v, 2, 1)
        _jacobi_rotate(h, v, 0, 1)

    w0, w1, w2 = h[(0, 0)], h[(1, 1)], h[(2, 2)]
    sel0 = jnp.logical_and(w0 <= w1, w0 <= w2)
    sel1 = jnp.logical_and(w1 < w0, w1 <= w2)
    n = []
    for i in range(3):
        n.append(jnp.where(sel0, v[(i, 0)],
                           jnp.where(sel1, v[(i, 1)], v[(i, 2)])))

    acc = jnp.zeros_like(n[0][:8, :])
    for i in range(3):
        dlt = n[i][:8, :] - n[i][8:, :]
        acc = acc + dlt * dlt
    out_ref[...] = jnp.zeros((8, 128), jnp.float32) + jnp.sum(acc)


@jax.jit
def _normals_mse(pts):
    g, _, n = pts.shape
    pts_t8 = jnp.concatenate(
        [pts, jnp.zeros((g, 5, n), dtype=pts.dtype)], axis=1)
    pts_n8 = jnp.transpose(pts_t8, (0, 2, 1))
    covs = pl.pallas_call(
        _cov_body,
        grid=(g, n // _R),
        in_specs=[
            pl.BlockSpec((1, 8, n), lambda b, i: (b, 0, 0)),
            pl.BlockSpec((1, n, 8), lambda b, i: (b, 0, 0)),
            pl.BlockSpec((1, _R, 8), lambda b, i: (b, i, 0)),
        ],
        out_specs=pl.BlockSpec((1, _R, 8), lambda b, i: (b, i, 0)),
        out_shape=jax.ShapeDtypeStruct((g, n, 8), jnp.float32),
    )(pts_t8, pts_n8, pts_n8)

    total = g * n
    cov6 = jnp.transpose(covs, (2, 0, 1)).reshape(8, 16, total // 16)[:6]
    sq_sum = pl.pallas_call(
        _eig_mse_body,
        out_shape=jax.ShapeDtypeStruct((8, 128), jnp.float32),
    )(cov6)
    return sq_sum[0, 0]


def kernel(pred, gt):
    b, c, n = pred.shape
    pts = jnp.concatenate([pred, gt], axis=0)
    return _normals_mse(pts) / (b * c * n)


# speedup vs baseline: 185.9492x; 1.8601x over previous
import functools

import jax
import jax.numpy as jnp
from jax.experimental import pallas as pl

_K = 10
_R = 512
_SWEEPS = 6


def _cov_body(ptsq_ref, p_ref, qpack_ref, out_ref):
    q8 = ptsq_ref[0]
    p = p_ref[0]
    qpack = qpack_ref[0]

    pq = jax.lax.dot_general(
        p, q8, (((1,), (0,)), ((), ())),
        preferred_element_type=jnp.float32)
    scores = q8[3:4, :] - 2.0 * pq

    m = jnp.min(scores, axis=1, keepdims=True)
    for _ in range(_K - 1):
        m = jnp.min(jnp.where(scores > m, scores, jnp.inf),
                    axis=1, keepdims=True)
    msum = jnp.where(scores <= m, 1.0, 0.0)

    mb = msum.astype(jnp.bfloat16)
    spack = jax.lax.dot_general(
        qpack, mb, (((0,), (1,)), ((), ())),
        preferred_element_type=jnp.float32)
    s12 = spack[0:16, :] + spack[16:32, :]

    inv_k = 1.0 / _K
    mux = s12[0:1, :] * inv_k
    muy = s12[1:2, :] * inv_k
    muz = s12[2:3, :] * inv_k
    cxx = s12[3:4, :] * inv_k - mux * mux
    cyy = s12[4:5, :] * inv_k - muy * muy
    czz = s12[5:6, :] * inv_k - muz * muz
    cxy = s12[6:7, :] * inv_k - mux * muy
    cxz = s12[7:8, :] * inv_k - mux * muz
    cyz = s12[8:9, :] * inv_k - muy * muz
    z0 = jnp.zeros_like(cxx)
    out_ref[0] = jnp.concatenate([cxx, cyy, czz, cxy, cxz, cyz, z0, z0],
                                 axis=0)


def _hk(i, j):
    return (min(i, j), max(i, j))


def _jacobi_rotate(h, v, p, q):
    r = 3 - p - q
    a = h[_hk(p, p)]
    b = h[_hk(q, q)]
    d = h[_hk(p, q)]
    e = h[_hk(p, r)]
    f = h[_hk(q, r)]
    tau = (b - a) / (2.0 * d)
    t = jnp.sign(tau) / (jnp.abs(tau) + jnp.sqrt(1.0 + tau * tau))
    t = jnp.where(d == 0.0, 0.0, t)
    c = jax.lax.rsqrt(1.0 + t * t)
    s = t * c
    cc = c * c
    ss = s * s
    sc2 = 2.0 * s * c
    h[_hk(p, p)] = cc * a - sc2 * d + ss * b
    h[_hk(q, q)] = ss * a + sc2 * d + cc * b
    h[_hk(p, q)] = s * c * (a - b) + (cc - ss) * d
    h[_hk(p, r)] = c * e - s * f
    h[_hk(q, r)] = s * e + c * f
    for i in range(3):
        vp = v[(i, p)]
        vq = v[(i, q)]
        v[(i, p)] = c * vp - s * vq
        v[(i, q)] = s * vp + c * vq


def _eig_mse_body(cov_ref, out_ref):
    h = {}
    h[(0, 0)] = cov_ref[0]
    h[(1, 1)] = cov_ref[1]
    h[(2, 2)] = cov_ref[2]
    h[(0, 1)] = cov_ref[3]
    h[(0, 2)] = cov_ref[4]
    h[(1, 2)] = cov_ref[5]
    one = jnp.ones_like(h[(0, 0)])
    zero = jnp.zeros_like(one)
    v = {}
    for i in range(3):
        for j in range(3):
            v[(i, j)] = one if i == j else zero

    for _ in range(_SWEEPS):
        _jacobi_rotate(h, v, 0, 2)
        _jacobi_rotate(h, v, 2, 1)
        _jacobi_rotate(h, v, 0, 1)

    w0, w1, w2 = h[(0, 0)], h[(1, 1)], h[(2, 2)]
    sel0 = jnp.logical_and(w0 <= w1, w0 <= w2)
    sel1 = jnp.logical_and(w1 < w0, w1 <= w2)
    n = []
    for i in range(3):
        n.append(jnp.where(sel0, v[(i, 0)],
                           jnp.where(sel1, v[(i, 1)], v[(i, 2)])))

    acc = jnp.zeros_like(n[0][:8, :])
    for i in range(3):
        dlt = n[i][:8, :] - n[i][8:, :]
        acc = acc + dlt * dlt
    out_ref[...] = jnp.zeros((8, 128), jnp.float32) + jnp.sum(acc)


@jax.jit
def _normals_mse(pts):
    g, _, n = pts.shape
    x = pts[:, 0]
    y = pts[:, 1]
    z = pts[:, 2]
    sq = x * x + y * y + z * z
    zn = jnp.zeros((g, n), dtype=pts.dtype)
    ptsq = jnp.stack([x, y, z, sq, zn, zn, zn, zn], axis=1)
    pts_n8 = jnp.transpose(
        jnp.stack([x, y, z, zn, zn, zn, zn, zn], axis=1), (0, 2, 1))
    q12 = jnp.stack(
        [x, y, z, x * x, y * y, z * z, x * y, x * z, y * z,
         zn, zn, zn, zn, zn, zn, zn], axis=1)
    q12 = jnp.transpose(q12, (0, 2, 1))
    qhi = q12.astype(jnp.bfloat16)
    qlo = (q12 - qhi.astype(jnp.float32)).astype(jnp.bfloat16)
    qpack = jnp.concatenate([qhi, qlo], axis=2)

    covs = pl.pallas_call(
        _cov_body,
        grid=(g, n // _R),
        in_specs=[
            pl.BlockSpec((1, 8, n), lambda b, i: (b, 0, 0)),
            pl.BlockSpec((1, _R, 8), lambda b, i: (b, i, 0)),
            pl.BlockSpec((1, n, 32), lambda b, i: (b, 0, 0)),
        ],
        out_specs=pl.BlockSpec((1, 8, _R), lambda b, i: (b, 0, i)),
        out_shape=jax.ShapeDtypeStruct((g, 8, n), jnp.float32),
    )(ptsq, pts_n8, qpack)

    total = g * n
    cov6 = jnp.transpose(covs, (1, 0, 2)).reshape(8, 16, total // 16)[:6]
    sq_sum = pl.pallas_call(
        _eig_mse_body,
        out_shape=jax.ShapeDtypeStruct((8, 128), jnp.float32),
    )(cov6)
    return sq_sum[0, 0]


def kernel(pred, gt):
    b, c, n = pred.shape
    pts = jnp.concatenate([pred, gt], axis=0)
    return _normals_mse(pts) / (b * c * n)
